# Initial kernel scaffold; baseline (speedup 1.0000x reference)
import jax
import jax.numpy as jnp
from jax.experimental import pallas as pl


def kernel(x, edge_index, mask, params):
    raise NotImplementedError("write your pallas kernel here")


# speedup vs baseline: 3.5019x; 3.5019x over previous
import functools

import jax
import jax.numpy as jnp
from jax import lax
from jax.experimental import pallas as pl
from jax.experimental.pallas import tpu as pltpu
from jax.experimental.pallas import tpu_sc as plsc

N_NODES = 10000
D = 128
N_CLASSES = 10
BN_EPS = 1e-5

NC = 2
NS = 16
NW = NC * NS

N_PAD = 10240
RPT = N_PAD // NS
CH = 128


def _seg_body(h_hbm, src_hbm, dst_hbm, zeros_hbm, out_hbm,
              sidx, didx, rows, acc, sem, *, ept):
    cid = lax.axis_index("c")
    sid = lax.axis_index("s")
    wid = cid * NS + sid
    pltpu.sync_copy(zeros_hbm, acc.at[pl.ds(sid * RPT, RPT)])
    plsc.subcore_barrier()
    ebase = wid * ept

    def body(g, carry):
        off = ebase + g * CH
        pltpu.sync_copy(src_hbm.at[pl.ds(off, CH)], sidx)
        pltpu.async_copy(h_hbm.at[sidx], rows, sem).wait()
        pltpu.sync_copy(dst_hbm.at[pl.ds(off, CH)], didx)
        pltpu.sync_copy(rows, acc.at[didx], add=True)
        return carry

    lax.fori_loop(0, ept // CH, body, 0)
    plsc.subcore_barrier()
    pltpu.sync_copy(acc.at[pl.ds(sid * RPT, RPT)],
                    out_hbm.at[pl.ds(cid * N_PAD + sid * RPT, RPT)])


def _segment_partials(h, src_p, dst_p, zeros, ept):
    mesh = plsc.VectorSubcoreMesh(core_axis_name="c", subcore_axis_name="s")
    kfn = pl.kernel(
        functools.partial(_seg_body, ept=ept),
        out_type=jax.ShapeDtypeStruct((NC * N_PAD, D), jnp.float32),
        mesh=mesh,
        scratch_types=[
            pltpu.VMEM((CH,), jnp.int32),
            pltpu.VMEM((CH,), jnp.int32),
            pltpu.VMEM((CH, D), jnp.float32),
            pltpu.VMEM_SHARED((N_PAD, D), jnp.float32),
            pltpu.SemaphoreType.DMA,
        ],
    )
    return kfn(h, src_p, dst_p, zeros)


def _linear_body(x_ref, w_ref, b_ref, o_ref):
    o_ref[...] = jnp.dot(x_ref[...], w_ref[...],
                         preferred_element_type=jnp.float32) + b_ref[...]


def _linear(x, w, b):
    n = x.shape[0]
    return pl.pallas_call(
        _linear_body,
        out_shape=jax.ShapeDtypeStruct((n, w.shape[1]), jnp.float32),
    )(x, w, b.reshape(1, -1))


def _bn(h, g, e):
    m = jnp.mean(h, axis=0, keepdims=True)
    v = jnp.mean(jnp.square(h - m), axis=0, keepdims=True)
    return (h - m) * (g * lax.rsqrt(v + BN_EPS)) + e


def _mlp_body(h_ref, p0_ref, p1_ref, w1_ref, b1_ref, g1_ref, e1_ref,
              w2_ref, b2_ref, g2_ref, e2_ref, o_ref):
    z = h_ref[...] + p0_ref[...] + p1_ref[...]
    h1 = jnp.dot(z, w1_ref[...], preferred_element_type=jnp.float32) + b1_ref[...]
    h1 = jnp.maximum(_bn(h1, g1_ref[...], e1_ref[...]), 0.0)
    h2 = jnp.dot(h1, w2_ref[...], preferred_element_type=jnp.float32) + b2_ref[...]
    o_ref[...] = jnp.maximum(_bn(h2, g2_ref[...], e2_ref[...]), 0.0)


def _mlp(h, p0, p1, p):
    n = h.shape[0]
    r = lambda a: a.reshape(1, -1)
    return pl.pallas_call(
        _mlp_body,
        out_shape=jax.ShapeDtypeStruct((n, p['W2'].shape[1]), jnp.float32),
    )(h, p0, p1, p['W1'], r(p['b1']), r(p['g1']), r(p['be1']),
      p['W2'], r(p['b2']), r(p['g2']), r(p['be2']))


def _final_body(h_ref, m_ref, w_ref, b_ref, o_ref):
    z = h_ref[...] * m_ref[...]
    o_ref[...] = jnp.dot(z, w_ref[...],
                         preferred_element_type=jnp.float32) + b_ref[...]


def _final(h, maskf, w, b):
    n = h.shape[0]
    return pl.pallas_call(
        _final_body,
        out_shape=jax.ShapeDtypeStruct((n, w.shape[1]), jnp.float32),
    )(h, maskf, w, b.reshape(1, -1))


def kernel(x, edge_index, mask, params):
    n = x.shape[0]
    e = edge_index.shape[1]
    src = edge_index[0].astype(jnp.int32)
    dst = edge_index[1].astype(jnp.int32)
    epw = NW * CH
    e_pad = ((e + epw - 1) // epw) * epw
    pad = e_pad - e
    if pad:
        src = jnp.concatenate([src, jnp.zeros((pad,), jnp.int32)])
        dst = jnp.concatenate([dst, jnp.full((pad,), N_PAD - 8, jnp.int32)])
    zeros = jnp.zeros((RPT, D), jnp.float32)

    h = _linear(x, params['init_W'], params['init_b'])
    for p in params['convs']:
        parts = _segment_partials(h, src, dst, zeros, e_pad // NW)
        h = _mlp(h, parts[0:n], parts[N_PAD:N_PAD + n], p)

    maskf = mask.astype(jnp.float32)[:, None]
    wp = jnp.pad(params['lin_W'], ((0, 0), (0, D - N_CLASSES)))
    bp = jnp.pad(params['lin_b'], (0, D - N_CLASSES))
    out = _final(h, maskf, wp, bp)
    return out[:, :N_CLASSES]
